# baseline (speedup 1.0000x reference)
import functools

import jax
import jax.numpy as jnp
from jax import lax
from jax.experimental import pallas as pl
from jax.experimental.pallas import tpu as pltpu
from jax.experimental.pallas import tpu_sc as plsc

_DIM = 32
_NW = 32
_NB = 4
_NBUF = 2
_SPLITS = ((0, 128), (128, 72))


def _make_gather(batch, hist):
    rows_per_w = batch // _NW
    n_chunks = rows_per_w // _NB
    assert n_chunks % _NBUF == 0
    mesh = plsc.VectorSubcoreMesh(core_axis_name="c", subcore_axis_name="s")

    @functools.partial(
        pl.kernel,
        out_type=jax.ShapeDtypeStruct((batch, hist, _DIM), jnp.float32),
        mesh=mesh,
        scratch_types=[
            pltpu.VMEM((_NBUF, _NB, hist), jnp.int32),
            pltpu.VMEM((_NBUF, _NB, hist, _DIM), jnp.float32),
            pltpu.SemaphoreType.DMA((_NBUF,)),
            pltpu.SemaphoreType.DMA((_NBUF,)),
            pltpu.SemaphoreType.DMA((_NBUF,)),
        ],
        compiler_params=pltpu.CompilerParams(use_tc_tiling_on_sc=False),
    )
    def gather_kernel(idx_hbm, table_hbm, out_hbm, idx_v, rows_v,
                      idx_sem, gat_sem, wb_sem):
        wid = lax.axis_index("s") * 2 + lax.axis_index("c")
        row0 = wid * rows_per_w

        def idx_copy(g, b):
            return pltpu.make_async_copy(
                idx_hbm.at[pl.ds(row0 + g * _NB, _NB)],
                idx_v.at[b], idx_sem.at[b])

        def wb_copy(g, b):
            return pltpu.make_async_copy(
                rows_v.at[b],
                out_hbm.at[pl.ds(row0 + g * _NB, _NB)],
                wb_sem.at[b])

        idx_copy(0, 0).start()

        def body(gg, carry):
            for b in range(_NBUF):
                g = gg * _NBUF + b
                idx_copy(g, b).wait()

                @pl.when(g + 1 < n_chunks)
                def _():
                    idx_copy(g + 1, (b + 1) % _NBUF).start()

                @pl.when(g >= _NBUF)
                def _():
                    wb_copy(g - _NBUF, b).wait()

                copies = [
                    pltpu.async_copy(
                        table_hbm.at[idx_v.at[b, i, pl.ds(lo, ln)]],
                        rows_v.at[b, i, pl.ds(lo, ln)],
                        gat_sem.at[b],
                    )
                    for i in range(_NB)
                    for (lo, ln) in _SPLITS
                ]
                for c in copies:
                    c.wait()
                wb_copy(g, b).start()
            return carry

        lax.fori_loop(0, n_chunks // _NBUF, body, 0)
        for b in range(_NBUF):
            wb_copy(n_chunks - _NBUF + b, b).wait()

    return gather_kernel


def kernel(indices, table):
    b, h = indices.shape
    return _make_gather(b, h)(indices, table)
